# Initial kernel scaffold; baseline (speedup 1.0000x reference)
import jax
import jax.numpy as jnp
from jax.experimental import pallas as pl


def kernel(x, edge_index, W, b):
    raise NotImplementedError("write your pallas kernel here")


# speedup vs baseline: 7.5656x; 7.5656x over previous
import functools

import jax
import jax.numpy as jnp
from jax import lax
from jax.experimental import pallas as pl
from jax.experimental.pallas import tpu as pltpu
from jax.experimental.pallas import tpu_sc as plsc

N = 10000
E = 320000
D = 128

NC = 2
NS = 16
NW = NC * NS

EPW = E // NW
CB = 80
NCHUNK = EPW // CB
GC = 8

ROWS_PT = N // 10
CNT_PT = N // 10


def _matmul_body(x_ref, wa_ref, wb_ref, b_ref, a_out, b_out):
    xb = x_ref[...]
    a_out[...] = jnp.dot(xb, wa_ref[...], preferred_element_type=jnp.float32) + b_ref[...]
    b_out[...] = jnp.dot(xb, wb_ref[...], preferred_element_type=jnp.float32)


def _node_features(x, wa, wb, b2d):
    return pl.pallas_call(
        _matmul_body,
        out_shape=(
            jax.ShapeDtypeStruct((N, D), jnp.float32),
            jax.ShapeDtypeStruct((N, D), jnp.float32),
        ),
    )(x, wa, wb, b2d)


def _edge_body(a_hbm, b_hbm, dst_hbm, src_hbm, p_hbm, cnt0_hbm, cnt1_hbm,
               acc, cnt_s, dst_v, src_v, arows, brows, ones_v, zcnt,
               sem_a, sem_b):
    cid = lax.axis_index("c")
    sid = lax.axis_index("s")

    def _zero_arows(r, _):
        for k in range(8):
            arows[r, pl.ds(k * 16, 16)] = jnp.zeros((16,), jnp.float32)
        return ()
    lax.fori_loop(0, CB, _zero_arows, (), unroll=False)

    def _zero_zcnt(i, _):
        zcnt[pl.ds(i * 16, 16)] = jnp.zeros((16,), jnp.float32)
        return ()
    lax.fori_loop(0, 63, _zero_zcnt, (), unroll=False)

    @pl.when(sid < 10)
    def _():
        for j in range(ROWS_PT // CB):
            base = sid * ROWS_PT + j * CB
            pltpu.sync_copy(arows, acc.at[pl.ds(base, CB)])
        pltpu.sync_copy(arows.at[pl.ds(0, 40)],
                        acc.at[pl.ds(sid * ROWS_PT + (ROWS_PT // CB) * CB, 40)])
        pltpu.sync_copy(zcnt.at[pl.ds(0, CNT_PT)],
                        cnt_s.at[pl.ds(sid * CNT_PT, CNT_PT)])

    for k in range(5):
        ones_v[pl.ds(k * 16, 16)] = jnp.ones((16,), jnp.float32)

    plsc.subcore_barrier()

    wid = cid * NS + sid

    def _chunk(c, _):
        idx_d = dst_v.at[c]
        idx_s = src_v.at[c]
        cp_a = pltpu.async_copy(a_hbm.at[idx_d], arows, sem_a)
        cp_b = pltpu.async_copy(b_hbm.at[idx_s], brows, sem_b)
        cp_a.wait()
        cp_b.wait()

        def _row(e, _):
            for k in range(8):
                sl = pl.ds(k * 16, 16)
                v = arows[e, sl] + brows[e, sl]
                arows[e, sl] = jnp.maximum(v, 0.0)
            return ()
        lax.fori_loop(0, CB, _row, (), unroll=False)

        pltpu.sync_copy(arows, acc.at[idx_d], add=True)
        pltpu.sync_copy(ones_v, cnt_s.at[idx_d], add=True)
        return ()

    def _group(g, _):
        pltpu.sync_copy(dst_hbm.at[wid, pl.ds(g * GC, GC)], dst_v)
        pltpu.sync_copy(src_hbm.at[wid, pl.ds(g * GC, GC)], src_v)
        lax.fori_loop(0, GC, _chunk, (), unroll=False)
        return ()
    lax.fori_loop(0, NCHUNK // GC, _group, (), unroll=False)

    tail = (NCHUNK // GC) * GC
    pltpu.sync_copy(dst_hbm.at[wid, pl.ds(tail, NCHUNK - tail)],
                    dst_v.at[pl.ds(0, NCHUNK - tail)])
    pltpu.sync_copy(src_hbm.at[wid, pl.ds(tail, NCHUNK - tail)],
                    src_v.at[pl.ds(0, NCHUNK - tail)])
    lax.fori_loop(0, NCHUNK - tail, _chunk, (), unroll=False)

    plsc.subcore_barrier()

    @pl.when(sid < 10)
    def _():
        for j in range(ROWS_PT // CB):
            base = sid * ROWS_PT + j * CB
            pltpu.sync_copy(acc.at[pl.ds(base, CB)], brows)
            pltpu.sync_copy(brows, p_hbm.at[cid, pl.ds(base, CB)])
        tail = sid * ROWS_PT + (ROWS_PT // CB) * CB
        pltpu.sync_copy(acc.at[pl.ds(tail, 40)], brows.at[pl.ds(0, 40)])
        pltpu.sync_copy(brows.at[pl.ds(0, 40)],
                        p_hbm.at[cid, pl.ds(tail, 40)])

        pltpu.sync_copy(cnt_s.at[pl.ds(sid * CNT_PT, CNT_PT)],
                        zcnt.at[pl.ds(0, CNT_PT)])

        @pl.when(cid == 0)
        def _():
            pltpu.sync_copy(zcnt.at[pl.ds(0, CNT_PT)],
                            cnt0_hbm.at[pl.ds(sid * CNT_PT, CNT_PT)])

        @pl.when(cid == 1)
        def _():
            pltpu.sync_copy(zcnt.at[pl.ds(0, CNT_PT)],
                            cnt1_hbm.at[pl.ds(sid * CNT_PT, CNT_PT)])


@functools.partial(
    pl.kernel,
    out_type=(
        jax.ShapeDtypeStruct((NC, N, D), jnp.float32),
        jax.ShapeDtypeStruct((N,), jnp.float32),
        jax.ShapeDtypeStruct((N,), jnp.float32),
    ),
    mesh=plsc.VectorSubcoreMesh(
        core_axis_name="c", subcore_axis_name="s", num_cores=NC, num_subcores=NS
    ),
    scratch_types=[
        pltpu.VMEM_SHARED((N, D), jnp.float32),
        pltpu.VMEM_SHARED((N,), jnp.float32),
        pltpu.VMEM((GC, CB), jnp.int32),
        pltpu.VMEM((GC, CB), jnp.int32),
        pltpu.VMEM((CB, D), jnp.float32),
        pltpu.VMEM((CB, D), jnp.float32),
        pltpu.VMEM((CB,), jnp.float32),
        pltpu.VMEM((1008,), jnp.float32),
        pltpu.SemaphoreType.DMA,
        pltpu.SemaphoreType.DMA,
    ],
)
def _edge_kernel(a_hbm, b_hbm, dst_hbm, src_hbm, p_hbm, cnt0_hbm, cnt1_hbm,
                 *scratch):
    _edge_body(a_hbm, b_hbm, dst_hbm, src_hbm, p_hbm, cnt0_hbm, cnt1_hbm,
               *scratch)


def _finalize_body(p_ref, c0_ref, c1_ref, o_ref):
    cnt = c0_ref[...] + c1_ref[...]
    inv = 1.0 / jnp.maximum(cnt, 1.0)
    o_ref[...] = (p_ref[0] + p_ref[1]) * inv[:, None]


def _finalize(p, cnt0, cnt1):
    return pl.pallas_call(
        _finalize_body,
        out_shape=jax.ShapeDtypeStruct((N, D), jnp.float32),
    )(p, cnt0, cnt1)


def kernel(x, edge_index, W, b):
    w1 = W[:, :D]
    w2 = W[:, D:]
    wa = (w1 - w2).T
    wb = w2.T
    b2d = b[None, :]
    a_nodes, b_nodes = _node_features(x, wa, wb, b2d)
    src = edge_index[0].reshape(NW, NCHUNK, CB)
    dst = edge_index[1].reshape(NW, NCHUNK, CB)
    p, cnt0, cnt1 = _edge_kernel(a_nodes, b_nodes, dst, src)
    return _finalize(p, cnt0, cnt1)
